# baseline (speedup 1.0000x reference)
import jax
import jax.numpy as jnp
from jax.experimental import pallas as pl


def _body(x_ref, y_ref, o_ref):
    yv = y_ref[...]
    xv = x_ref[...]
    inv = 1.0 / yv
    o_ref[...] = jnp.where(yv != 0.0, inv * xv, 0.0)


def kernel(x, y):
    N = x.shape[0]
    BLK = 524288
    out = pl.pallas_call(
        _body,
        grid=(N // BLK,),
        in_specs=[
            pl.BlockSpec((BLK,), lambda i: (i,)),
            pl.BlockSpec((BLK,), lambda i: (i,)),
        ],
        out_specs=pl.BlockSpec((BLK,), lambda i: (i,)),
        out_shape=jax.ShapeDtypeStruct((N,), jnp.float32),
    )(x, y)
    return out


# speedup vs baseline: 1.0249x; 1.0249x over previous
import jax
import jax.numpy as jnp
from jax.experimental import pallas as pl


def _body(x_ref, y_ref, o_ref):
    yv = y_ref[...]
    xv = x_ref[...]
    inv = 1.0 / yv
    o_ref[...] = jnp.where(yv != 0.0, inv * xv, 0.0)


def kernel(x, y):
    N = x.shape[0]
    BLK = 1048576
    out = pl.pallas_call(
        _body,
        grid=(N // BLK,),
        in_specs=[
            pl.BlockSpec((BLK,), lambda i: (i,)),
            pl.BlockSpec((BLK,), lambda i: (i,)),
        ],
        out_specs=pl.BlockSpec((BLK,), lambda i: (i,)),
        out_shape=jax.ShapeDtypeStruct((N,), jnp.float32),
    )(x, y)
    return out
